# baseline (speedup 1.0000x reference)
import functools

import jax
import jax.numpy as jnp
from jax import lax
from jax.experimental import pallas as pl
from jax.experimental.pallas import tpu as pltpu
from jax.experimental.pallas import tpu_sc as plsc

_DIM = 64
_LANES = 16
_CHUNK = 128
_STEP = 16


def _rsqrt_vec(x):
    s = jnp.full((_LANES,), 8.0, dtype=jnp.float32)
    for _ in range(6):
        s = 0.5 * (s + x / s)
    return 1.0 / s


def _lane_sum(x):
    for s in (8, 4, 2, 1):
        perm = lax.iota(jnp.int32, _LANES) ^ s
        x = x + x.at[perm].get(mode="promise_in_bounds")
    return x


def kernel(user_ids, item_ids, user_table, item_table):
    info = plsc.get_sparse_core_info()
    nc, ns = info.num_cores, info.num_subcores
    nw = nc * ns
    batch = user_ids.shape[0]
    b_per_w = batch // nw
    n_chunks = b_per_w // _CHUNK
    n_steps = b_per_w // _STEP

    mesh = plsc.VectorSubcoreMesh(core_axis_name="c", subcore_axis_name="s")

    @functools.partial(
        pl.kernel,
        mesh=mesh,
        out_type=(
            jax.ShapeDtypeStruct((batch // 2, 2 * _DIM), jnp.float32),
            jax.ShapeDtypeStruct((batch // 2, 2 * _DIM), jnp.float32),
        ),
        scratch_types=[
            pltpu.VMEM((n_chunks, _CHUNK), jnp.int32),
            pltpu.VMEM((n_chunks, _CHUNK), jnp.int32),
            pltpu.VMEM((n_chunks, _CHUNK), jnp.int32),
            pltpu.VMEM((b_per_w, 2 * _DIM), jnp.float32),
            pltpu.VMEM((b_per_w // 2, 2 * _DIM), jnp.float32),
            pltpu.SemaphoreType.DMA,
            pltpu.SemaphoreType.DMA,
        ],
    )
    def sc_lookup(uidx_hbm, iidx_hbm, utab_hbm, itab_hbm, uout_hbm, iout_hbm,
                  uidx_v, iidx_v, pidx_v, raw, outacc, sem_in, sem_out):
        wid = lax.axis_index("s") * nc + lax.axis_index("c")
        base = wid * b_per_w
        pltpu.sync_copy(uidx_hbm.at[wid], uidx_v)
        pltpu.sync_copy(iidx_hbm.at[wid], iidx_v)

        def process_table(tab_hbm, idx_v, out_hbm):
            for c in range(n_chunks):
                for v0 in range(0, _CHUNK, _LANES):
                    pidx_v[c, pl.ds(v0, _LANES)] = (
                        idx_v[c, pl.ds(v0, _LANES)] >> 1
                    )
            cps = [
                pltpu.async_copy(
                    tab_hbm.at[pidx_v.at[c]],
                    raw.at[pl.ds(c * _CHUNK, _CHUNK)],
                    sem_in,
                )
                for c in range(n_chunks)
            ]
            for cp in cps:
                cp.wait()

            def step(g, carry):
                gbase = g * _STEP
                idvec = idx_v[g >> 3, pl.ds((g & 7) * _STEP, _STEP)]
                for k in range(_STEP):
                    idk = idvec[k]
                    half = (idk & 1) * _DIM
                    row = gbase + k
                    vs = [
                        raw[row, pl.ds(half + j0, _LANES)]
                        for j0 in range(0, _DIM, _LANES)
                    ]
                    ss = _lane_sum(
                        vs[0] * vs[0] + vs[1] * vs[1]
                        + vs[2] * vs[2] + vs[3] * vs[3]
                    )
                    y = _rsqrt_vec(jnp.maximum(ss, 1e-24))
                    prow = g * (_STEP // 2) + (k >> 1)
                    hoff = (k & 1) * _DIM
                    for j, v in enumerate(vs):
                        outacc[prow, pl.ds(hoff + j * _LANES, _LANES)] = v * y
                pltpu.async_copy(
                    outacc.at[pl.ds(g * (_STEP // 2), _STEP // 2)],
                    out_hbm.at[
                        pl.ds(
                            pl.multiple_of(
                                base // 2 + g * (_STEP // 2), _STEP // 2
                            ),
                            _STEP // 2,
                        )
                    ],
                    sem_out,
                )
                return carry

            lax.fori_loop(0, n_steps, step, 0)
            pltpu.make_async_copy(
                out_hbm.at[
                    pl.ds(pl.multiple_of(base // 2, 8), b_per_w // 2)
                ],
                outacc,
                sem_out,
            ).wait()

        process_table(utab_hbm, uidx_v, uout_hbm)
        process_table(itab_hbm, iidx_v, iout_hbm)

    nw_ids_u = user_ids.astype(jnp.int32).reshape(nw, n_chunks, _CHUNK)
    nw_ids_i = item_ids.astype(jnp.int32).reshape(nw, n_chunks, _CHUNK)
    pt_u = user_table.reshape(user_table.shape[0] // 2, 2 * _DIM)
    pt_i = item_table.reshape(item_table.shape[0] // 2, 2 * _DIM)
    u_pairs, i_pairs = sc_lookup(nw_ids_u, nw_ids_i, pt_u, pt_i)
    return u_pairs.reshape(batch, _DIM), i_pairs.reshape(batch, _DIM)


# speedup vs baseline: 1.0597x; 1.0597x over previous
import functools

import jax
import jax.numpy as jnp
from jax import lax
from jax.experimental import pallas as pl
from jax.experimental.pallas import tpu as pltpu
from jax.experimental.pallas import tpu_sc as plsc

_DIM = 64
_LANES = 16
_CHUNK = 128
_UNROLL = 8


def _rsqrt_vec(x):
    s = jnp.full((_LANES,), 8.0, dtype=jnp.float32)
    for _ in range(6):
        s = 0.5 * (s + x / s)
    return 1.0 / s


def _lane_sum(x):
    for s in (8, 4, 2, 1):
        perm = lax.iota(jnp.int32, _LANES) ^ s
        x = x + x.at[perm].get(mode="promise_in_bounds")
    return x


def _normalize_chunk(rows, start):
    def body(g, carry):
        for k in range(_UNROLL):
            i = start + g * _UNROLL + k
            v0 = rows[i, pl.ds(0, _LANES)]
            v1 = rows[i, pl.ds(_LANES, _LANES)]
            v2 = rows[i, pl.ds(2 * _LANES, _LANES)]
            v3 = rows[i, pl.ds(3 * _LANES, _LANES)]
            ss = _lane_sum(v0 * v0 + v1 * v1 + v2 * v2 + v3 * v3)
            ssv = jnp.maximum(ss, 1e-24)
            y = _rsqrt_vec(ssv)
            rows[i, pl.ds(0, _LANES)] = v0 * y
            rows[i, pl.ds(_LANES, _LANES)] = v1 * y
            rows[i, pl.ds(2 * _LANES, _LANES)] = v2 * y
            rows[i, pl.ds(3 * _LANES, _LANES)] = v3 * y
        return carry

    lax.fori_loop(0, _CHUNK // _UNROLL, body, 0)


def kernel(user_ids, item_ids, user_table, item_table):
    info = plsc.get_sparse_core_info()
    nc, ns = info.num_cores, info.num_subcores
    nw = nc * ns
    batch = user_ids.shape[0]
    b_per_w = batch // nw
    n_chunks = b_per_w // _CHUNK

    mesh = plsc.VectorSubcoreMesh(core_axis_name="c", subcore_axis_name="s")

    @functools.partial(
        pl.kernel,
        mesh=mesh,
        compiler_params=pltpu.CompilerParams(use_tc_tiling_on_sc=False),
        out_type=(
            jax.ShapeDtypeStruct((batch, _DIM), jnp.float32),
            jax.ShapeDtypeStruct((batch, _DIM), jnp.float32),
        ),
        scratch_types=[
            pltpu.VMEM((n_chunks, _CHUNK), jnp.int32),
            pltpu.VMEM((n_chunks, _CHUNK), jnp.int32),
            pltpu.VMEM((b_per_w, _DIM), jnp.float32),
            pltpu.VMEM((b_per_w, _DIM), jnp.float32),
            pltpu.SemaphoreType.DMA,
            pltpu.SemaphoreType.DMA,
            pltpu.SemaphoreType.DMA,
        ],
    )
    def sc_lookup(uidx_hbm, iidx_hbm, utab_hbm, itab_hbm, uout_hbm, iout_hbm,
                  uidx_v, iidx_v, urows, irows, sem_u, sem_i, sem_out):
        wid = lax.axis_index("s") * nc + lax.axis_index("c")
        base = wid * b_per_w
        pltpu.sync_copy(uidx_hbm.at[wid], uidx_v)
        pltpu.sync_copy(iidx_hbm.at[wid], iidx_v)

        u_cp = [
            pltpu.async_copy(
                utab_hbm.at[uidx_v.at[c]],
                urows.at[pl.ds(c * _CHUNK, _CHUNK)],
                sem_u,
            )
            for c in range(n_chunks)
        ]
        i_cp = [
            pltpu.async_copy(
                itab_hbm.at[iidx_v.at[c]],
                irows.at[pl.ds(c * _CHUNK, _CHUNK)],
                sem_i,
            )
            for c in range(n_chunks)
        ]

        out_cp = []
        for c in range(n_chunks):
            u_cp[c].wait()
            _normalize_chunk(urows, c * _CHUNK)
            out_cp.append(
                pltpu.async_copy(
                    urows.at[pl.ds(c * _CHUNK, _CHUNK)],
                    uout_hbm.at[pl.ds(base + c * _CHUNK, _CHUNK)],
                    sem_out,
                )
            )
        for c in range(n_chunks):
            i_cp[c].wait()
            _normalize_chunk(irows, c * _CHUNK)
            out_cp.append(
                pltpu.async_copy(
                    irows.at[pl.ds(c * _CHUNK, _CHUNK)],
                    iout_hbm.at[pl.ds(base + c * _CHUNK, _CHUNK)],
                    sem_out,
                )
            )
        for cp in out_cp:
            cp.wait()

    nw_ids_u = user_ids.astype(jnp.int32).reshape(nw, n_chunks, _CHUNK)
    nw_ids_i = item_ids.astype(jnp.int32).reshape(nw, n_chunks, _CHUNK)
    return sc_lookup(nw_ids_u, nw_ids_i, user_table, item_table)
